# Initial kernel scaffold; baseline (speedup 1.0000x reference)
import jax
import jax.numpy as jnp
from jax.experimental import pallas as pl


def kernel(x, table, W, b):
    raise NotImplementedError("write your pallas kernel here")


# speedup vs baseline: 77.6171x; 77.6171x over previous
import jax
import jax.numpy as jnp
from jax import lax
from jax.experimental import pallas as pl
from jax.experimental.pallas import tpu as pltpu

N_BITS = 32
EMB_DIM = 64
OUT_DIM = 64
BATCH = 16384
FAN_IN = N_BITS * EMB_DIM
BLOCK = 2048


def _enc_kernel(x_ref, table_ref, w_ref, b_ref, o_ref, mt_ref, c_ref):
    @pl.when(pl.program_id(0) == 0)
    def _fold():
        t0 = table_ref[0:1, :]
        d = table_ref[1:2, :] - t0
        w = w_ref[...]
        te = lax.broadcasted_iota(jnp.int32, (EMB_DIM, FAN_IN), 0)
        tk = lax.broadcasted_iota(jnp.int32, (EMB_DIM, FAN_IN), 1)
        tile_mat = (tk % EMB_DIM == te).astype(jnp.float32)
        d_t = jnp.dot(d, tile_mat, preferred_element_type=jnp.float32)
        t0_t = jnp.dot(t0, tile_mat, preferred_element_type=jnp.float32)
        wd = w * d_t
        sk = lax.broadcasted_iota(jnp.int32, (FAN_IN, N_BITS), 0)
        sj = lax.broadcasted_iota(jnp.int32, (FAN_IN, N_BITS), 1)
        sel = (sk // EMB_DIM == sj).astype(jnp.float32)
        mt_ref[...] = jnp.dot(wd, sel, preferred_element_type=jnp.float32)
        c_ref[...] = b_ref[...] + lax.dot_general(
            t0_t, w, (((1,), (1,)), ((), ())),
            preferred_element_type=jnp.float32)

    xf = x_ref[...].astype(jnp.float32)
    o_ref[...] = lax.dot_general(
        xf, mt_ref[...], (((1,), (1,)), ((), ())),
        preferred_element_type=jnp.float32) + c_ref[...]


def kernel(x, table, W, b):
    b2 = b.reshape(1, OUT_DIM)
    grid = (BATCH // BLOCK,)
    return pl.pallas_call(
        _enc_kernel,
        grid=grid,
        in_specs=[
            pl.BlockSpec((BLOCK, N_BITS), lambda i: (i, 0)),
            pl.BlockSpec((2, EMB_DIM), lambda i: (0, 0)),
            pl.BlockSpec((OUT_DIM, FAN_IN), lambda i: (0, 0)),
            pl.BlockSpec((1, OUT_DIM), lambda i: (0, 0)),
        ],
        out_specs=pl.BlockSpec((BLOCK, OUT_DIM), lambda i: (i, 0)),
        out_shape=jax.ShapeDtypeStruct((BATCH, OUT_DIM), jnp.float32),
        scratch_shapes=[
            pltpu.VMEM((OUT_DIM, N_BITS), jnp.float32),
            pltpu.VMEM((1, OUT_DIM), jnp.float32),
        ],
    )(x, table, W, b2)


# speedup vs baseline: 86.3508x; 1.1125x over previous
import jax
import jax.numpy as jnp
from jax import lax
from jax.experimental import pallas as pl
from jax.experimental.pallas import tpu as pltpu

N_BITS = 32
EMB_DIM = 64
OUT_DIM = 64
BATCH = 16384
FAN_IN = N_BITS * EMB_DIM
BLOCK = 4096


def _enc_kernel(x_ref, table_ref, w_ref, b_ref, o_ref, mt_ref, c_ref):
    @pl.when(pl.program_id(0) == 0)
    def _fold():
        t0 = table_ref[0:1, :]
        d = table_ref[1:2, :] - t0
        w = w_ref[...]
        te = lax.broadcasted_iota(jnp.int32, (EMB_DIM, FAN_IN), 0)
        tk = lax.broadcasted_iota(jnp.int32, (EMB_DIM, FAN_IN), 1)
        tile_mat = (tk % EMB_DIM == te).astype(jnp.float32)
        d_t = jnp.dot(d, tile_mat, preferred_element_type=jnp.float32)
        t0_t = jnp.dot(t0, tile_mat, preferred_element_type=jnp.float32)
        wd = w * d_t
        sk = lax.broadcasted_iota(jnp.int32, (FAN_IN, N_BITS), 0)
        sj = lax.broadcasted_iota(jnp.int32, (FAN_IN, N_BITS), 1)
        sel = (sk // EMB_DIM == sj).astype(jnp.float32)
        mt_ref[...] = jnp.dot(wd, sel, preferred_element_type=jnp.float32)
        c_ref[...] = b_ref[...] + lax.dot_general(
            t0_t, w, (((1,), (1,)), ((), ())),
            preferred_element_type=jnp.float32)

    xf = x_ref[...].astype(jnp.float32)
    o_ref[...] = lax.dot_general(
        xf, mt_ref[...], (((1,), (1,)), ((), ())),
        preferred_element_type=jnp.float32) + c_ref[...]


def kernel(x, table, W, b):
    b2 = b.reshape(1, OUT_DIM)
    grid = (BATCH // BLOCK,)
    return pl.pallas_call(
        _enc_kernel,
        grid=grid,
        in_specs=[
            pl.BlockSpec((BLOCK, N_BITS), lambda i: (i, 0)),
            pl.BlockSpec((2, EMB_DIM), lambda i: (0, 0)),
            pl.BlockSpec((OUT_DIM, FAN_IN), lambda i: (0, 0)),
            pl.BlockSpec((1, OUT_DIM), lambda i: (0, 0)),
        ],
        out_specs=pl.BlockSpec((BLOCK, OUT_DIM), lambda i: (i, 0)),
        out_shape=jax.ShapeDtypeStruct((BATCH, OUT_DIM), jnp.float32),
        scratch_shapes=[
            pltpu.VMEM((OUT_DIM, N_BITS), jnp.float32),
            pltpu.VMEM((1, OUT_DIM), jnp.float32),
        ],
    )(x, table, W, b2)


# speedup vs baseline: 92.0115x; 1.0656x over previous
import jax
import jax.numpy as jnp
from jax import lax
from jax.experimental import pallas as pl
from jax.experimental.pallas import tpu as pltpu

N_BITS = 32
EMB_DIM = 64
OUT_DIM = 64
BATCH = 16384
FAN_IN = N_BITS * EMB_DIM
BLOCK = 8192


def _enc_kernel(x_ref, table_ref, w_ref, b_ref, o_ref, mt_ref, c_ref):
    @pl.when(pl.program_id(0) == 0)
    def _fold():
        t0 = table_ref[0:1, :]
        d = table_ref[1:2, :] - t0
        w = w_ref[...]
        te = lax.broadcasted_iota(jnp.int32, (EMB_DIM, FAN_IN), 0)
        tk = lax.broadcasted_iota(jnp.int32, (EMB_DIM, FAN_IN), 1)
        tile_mat = (tk % EMB_DIM == te).astype(jnp.float32)
        d_t = jnp.dot(d, tile_mat, preferred_element_type=jnp.float32)
        t0_t = jnp.dot(t0, tile_mat, preferred_element_type=jnp.float32)
        wd = w * d_t
        sk = lax.broadcasted_iota(jnp.int32, (FAN_IN, N_BITS), 0)
        sj = lax.broadcasted_iota(jnp.int32, (FAN_IN, N_BITS), 1)
        sel = (sk // EMB_DIM == sj).astype(jnp.float32)
        mt_ref[...] = jnp.dot(wd, sel, preferred_element_type=jnp.float32)
        c_ref[...] = b_ref[...] + lax.dot_general(
            t0_t, w, (((1,), (1,)), ((), ())),
            preferred_element_type=jnp.float32)

    xf = x_ref[...].astype(jnp.float32)
    o_ref[...] = lax.dot_general(
        xf, mt_ref[...], (((1,), (1,)), ((), ())),
        preferred_element_type=jnp.float32) + c_ref[...]


def kernel(x, table, W, b):
    b2 = b.reshape(1, OUT_DIM)
    grid = (BATCH // BLOCK,)
    return pl.pallas_call(
        _enc_kernel,
        grid=grid,
        in_specs=[
            pl.BlockSpec((BLOCK, N_BITS), lambda i: (i, 0)),
            pl.BlockSpec((2, EMB_DIM), lambda i: (0, 0)),
            pl.BlockSpec((OUT_DIM, FAN_IN), lambda i: (0, 0)),
            pl.BlockSpec((1, OUT_DIM), lambda i: (0, 0)),
        ],
        out_specs=pl.BlockSpec((BLOCK, OUT_DIM), lambda i: (i, 0)),
        out_shape=jax.ShapeDtypeStruct((BATCH, OUT_DIM), jnp.float32),
        scratch_shapes=[
            pltpu.VMEM((OUT_DIM, N_BITS), jnp.float32),
            pltpu.VMEM((1, OUT_DIM), jnp.float32),
        ],
    )(x, table, W, b2)
